# baseline (speedup 1.0000x reference)
import functools

import jax
import jax.numpy as jnp
from jax import lax
from jax.experimental import pallas as pl
from jax.experimental.pallas import tpu as pltpu
from jax.experimental.pallas import tpu_sc as plsc

BATCH = 16384
HIST = 200
HIDDEN = 64

NC, NS = 2, 16
NW = NC * NS
SUB = 128
B = BATCH * HIST
IDX_PER_W = B // NW
CH = 256
NCHUNK = IDX_PER_W // CH
NROWS = B // CH
GATHER_SPLITS = [(0, 128), (128, 128)]

_mesh = plsc.VectorSubcoreMesh(core_axis_name="c", subcore_axis_name="s")


@functools.partial(
    pl.kernel,
    out_type=jax.ShapeDtypeStruct((B, HIDDEN), jnp.float32),
    mesh=_mesh,
    scratch_types=[
        pltpu.VMEM((2, 1, CH), jnp.int32),
        pltpu.VMEM((2, CH, HIDDEN), jnp.float32),
        pltpu.SemaphoreType.DMA,
        pltpu.SemaphoreType.DMA,
        pltpu.SemaphoreType.DMA,
        pltpu.SemaphoreType.DMA,
        pltpu.SemaphoreType.DMA,
        pltpu.SemaphoreType.DMA,
    ],
    compiler_params=pltpu.CompilerParams(use_tc_tiling_on_sc=False),
)
def _emb_lookup(idx_hbm, table_hbm, out_hbm, idx_v, rows_v,
                is0, is1, gs0, gs1, os0, os1):
  isems = (is0, is1)
  gsems = (gs0, gs1)
  osems = (os0, os1)
  wid = lax.axis_index("s") * NC + lax.axis_index("c")
  wbase = wid * IDX_PER_W
  wrow = wid * NCHUNK

  def idx_copy(i, b):
    return pltpu.make_async_copy(
        idx_hbm.at[pl.ds(wrow + i, 1)], idx_v.at[b], isems[b])

  def gather_copies(b):
    return [
        pltpu.make_async_copy(
            table_hbm.at[idx_v.at[b, 0, pl.ds(o, n)]],
            rows_v.at[b, pl.ds(o, n)], gsems[b])
        for o, n in GATHER_SPLITS
    ]

  def out_copy(i, b):
    return pltpu.make_async_copy(
        rows_v.at[b],
        out_hbm.at[pl.ds(wbase + i * CH, CH)], osems[b])

  idx_copy(0, 0).start()

  def body(g, carry):
    for b in range(2):
      i = 2 * g + b
      idx_copy(i, b).wait()

      @pl.when(i >= 2)
      def _():
        out_copy(i - 2, b).wait()

      for cp in gather_copies(b):
        cp.start()

      @pl.when(i + 1 < NCHUNK)
      def _():
        idx_copy(i + 1, 1 - b).start()

      for cp in gather_copies(b):
        cp.wait()
      out_copy(i, b).start()
    return carry

  lax.fori_loop(0, NCHUNK // 2, body, 0)
  out_copy(NCHUNK - 2, 0).wait()
  out_copy(NCHUNK - 1, 1).wait()


def kernel(week_numbers, week_embed):
  idx = week_numbers.astype(jnp.int32).reshape(NROWS, CH)
  out = _emb_lookup(idx, week_embed)
  return out.reshape(BATCH, HIST, HIDDEN)


# speedup vs baseline: 1.3907x; 1.3907x over previous
import functools

import jax
import jax.numpy as jnp
from jax import lax
from jax.experimental import pallas as pl
from jax.experimental.pallas import tpu as pltpu
from jax.experimental.pallas import tpu_sc as plsc

BATCH = 16384
HIST = 200
HIDDEN = 64

NC, NS = 2, 16
NW = NC * NS
SUB = 128
B = BATCH * HIST
IDX_PER_W = B // NW
CH = 256
NCHUNK = IDX_PER_W // CH
NROWS = B // CH
GATHER_SPLITS = [(0, 128), (128, 128)]

_mesh = plsc.VectorSubcoreMesh(core_axis_name="c", subcore_axis_name="s")


@functools.partial(
    pl.kernel,
    out_type=jax.ShapeDtypeStruct((B, HIDDEN), jnp.float32),
    mesh=_mesh,
    scratch_types=[
        pltpu.VMEM((2, 1, CH), jnp.int32),
        pltpu.VMEM((2, CH, HIDDEN), jnp.float32),
        pltpu.VMEM_SHARED((1000, HIDDEN), jnp.float32),
        pltpu.SemaphoreType.DMA,
        pltpu.SemaphoreType.DMA,
        pltpu.SemaphoreType.DMA,
        pltpu.SemaphoreType.DMA,
        pltpu.SemaphoreType.DMA,
        pltpu.SemaphoreType.DMA,
    ],
    compiler_params=pltpu.CompilerParams(use_tc_tiling_on_sc=False),
)
def _emb_lookup(idx_hbm, table_hbm, out_hbm, idx_v, rows_v, table_s,
                is0, is1, gs0, gs1, os0, os1):
  isems = (is0, is1)
  gsems = (gs0, gs1)
  osems = (os0, os1)
  sid = lax.axis_index("s")
  wid = sid * NC + lax.axis_index("c")
  wbase = wid * IDX_PER_W
  wrow = wid * NCHUNK

  @pl.when(sid == 0)
  def _():
    pltpu.sync_copy(table_hbm, table_s)

  plsc.subcore_barrier()

  def idx_copy(i, b):
    return pltpu.make_async_copy(
        idx_hbm.at[pl.ds(wrow + i, 1)], idx_v.at[b], isems[b])

  def gather_copies(b):
    return [
        pltpu.make_async_copy(
            table_s.at[idx_v.at[b, 0, pl.ds(o, n)]],
            rows_v.at[b, pl.ds(o, n)], gsems[b])
        for o, n in GATHER_SPLITS
    ]

  def out_copy(i, b):
    return pltpu.make_async_copy(
        rows_v.at[b],
        out_hbm.at[pl.ds(wbase + i * CH, CH)], osems[b])

  idx_copy(0, 0).start()

  def body(g, carry):
    for b in range(2):
      i = 2 * g + b
      idx_copy(i, b).wait()

      @pl.when(i >= 2)
      def _():
        out_copy(i - 2, b).wait()

      for cp in gather_copies(b):
        cp.start()

      @pl.when(i + 1 < NCHUNK)
      def _():
        idx_copy(i + 1, 1 - b).start()

      for cp in gather_copies(b):
        cp.wait()
      out_copy(i, b).start()
    return carry

  lax.fori_loop(0, NCHUNK // 2, body, 0)
  out_copy(NCHUNK - 2, 0).wait()
  out_copy(NCHUNK - 1, 1).wait()


def kernel(week_numbers, week_embed):
  idx = week_numbers.astype(jnp.int32).reshape(NROWS, CH)
  out = _emb_lookup(idx, week_embed)
  return out.reshape(BATCH, HIST, HIDDEN)


# speedup vs baseline: 1.4030x; 1.0089x over previous
import functools

import jax
import jax.numpy as jnp
from jax import lax
from jax.experimental import pallas as pl
from jax.experimental.pallas import tpu as pltpu
from jax.experimental.pallas import tpu_sc as plsc

BATCH = 16384
HIST = 200
HIDDEN = 64

NC, NS = 2, 16
NW = NC * NS
SUB = 128
B = BATCH * HIST
IDX_PER_W = B // NW
CH = 512
NCHUNK = IDX_PER_W // CH
NROWS = B // CH
GATHER_SPLITS = [(0, 128), (128, 128), (256, 128), (384, 128)]

_mesh = plsc.VectorSubcoreMesh(core_axis_name="c", subcore_axis_name="s")


@functools.partial(
    pl.kernel,
    out_type=jax.ShapeDtypeStruct((B, HIDDEN), jnp.float32),
    mesh=_mesh,
    scratch_types=[
        pltpu.VMEM((2, 1, CH), jnp.int32),
        pltpu.VMEM((2, CH, HIDDEN), jnp.float32),
        pltpu.VMEM_SHARED((1000, HIDDEN), jnp.float32),
        pltpu.SemaphoreType.DMA,
        pltpu.SemaphoreType.DMA,
        pltpu.SemaphoreType.DMA,
        pltpu.SemaphoreType.DMA,
        pltpu.SemaphoreType.DMA,
        pltpu.SemaphoreType.DMA,
    ],
    compiler_params=pltpu.CompilerParams(use_tc_tiling_on_sc=False),
)
def _emb_lookup(idx_hbm, table_hbm, out_hbm, idx_v, rows_v, table_s,
                is0, is1, gs0, gs1, os0, os1):
  isems = (is0, is1)
  gsems = (gs0, gs1)
  osems = (os0, os1)
  sid = lax.axis_index("s")
  wid = sid * NC + lax.axis_index("c")
  wbase = wid * IDX_PER_W
  wrow = wid * NCHUNK

  @pl.when(sid == 0)
  def _():
    pltpu.sync_copy(table_hbm, table_s)

  plsc.subcore_barrier()

  def idx_copy(i, b):
    return pltpu.make_async_copy(
        idx_hbm.at[pl.ds(wrow + i, 1)], idx_v.at[b], isems[b])

  def gather_copies(b):
    return [
        pltpu.make_async_copy(
            table_s.at[idx_v.at[b, 0, pl.ds(o, n)]],
            rows_v.at[b, pl.ds(o, n)], gsems[b])
        for o, n in GATHER_SPLITS
    ]

  def out_copy(i, b):
    return pltpu.make_async_copy(
        rows_v.at[b],
        out_hbm.at[pl.ds(wbase + i * CH, CH)], osems[b])

  idx_copy(0, 0).start()

  def body(g, carry):
    for b in range(2):
      i = 2 * g + b
      idx_copy(i, b).wait()

      @pl.when(i >= 2)
      def _():
        out_copy(i - 2, b).wait()

      for cp in gather_copies(b):
        cp.start()

      @pl.when(i + 1 < NCHUNK)
      def _():
        idx_copy(i + 1, 1 - b).start()

      for cp in gather_copies(b):
        cp.wait()
      out_copy(i, b).start()
    return carry

  lax.fori_loop(0, NCHUNK // 2, body, 0)
  out_copy(NCHUNK - 2, 0).wait()
  out_copy(NCHUNK - 1, 1).wait()


def kernel(week_numbers, week_embed):
  idx = week_numbers.astype(jnp.int32).reshape(NROWS, CH)
  out = _emb_lookup(idx, week_embed)
  return out.reshape(BATCH, HIST, HIDDEN)
